# baseline (speedup 1.0000x reference)
import functools

import jax
import jax.numpy as jnp
import numpy as np
from jax.experimental import pallas as pl
from jax.experimental.pallas import tpu as pltpu

_N_BINS = 15
_DELTA = float(np.float32(1.0) / np.float32(_N_BINS))


def _ece_tc_kernel(labels_ref, logits_ref, ece_ref, acc_ref, *, nb_total,
                   n_samples):
    nb = pl.program_id(1)

    @pl.when(nb == 0)
    def _init():
        acc_ref[...] = jnp.zeros_like(acc_ref)

    x = logits_ref[0]
    c_dim, r_dim = x.shape
    m = jnp.max(x, axis=0, keepdims=True)
    e = jnp.exp(x - m)
    s = jnp.sum(e, axis=0, keepdims=True)
    conf = 1.0 / s

    iota = jax.lax.broadcasted_iota(jnp.int32, (c_dim, r_dim), 0)
    big = jnp.int32(2**30)
    fidx = jnp.min(jnp.where(x == m, iota, big), axis=0, keepdims=True)
    labels = labels_ref[0]
    correct = (fidx == labels).astype(jnp.float32)
    ones = jnp.ones((1, r_dim), jnp.float32)

    for i in range(_N_BINS):
        lo = -1.0 if i == 0 else float(np.float32(i) * np.float32(_DELTA))
        up = float(np.float32(i + 1) * np.float32(_DELTA))
        in_bin = (conf > lo) & (conf <= up)
        acc_ref[i:i + 1, :] += jnp.where(in_bin, conf, 0.0)
        acc_ref[16 + i:17 + i, :] += jnp.where(in_bin, correct, 0.0)
        acc_ref[32 + i:33 + i, :] += jnp.where(in_bin, ones, 0.0)

    @pl.when(nb == nb_total - 1)
    def _finish():
        conf_s = jnp.sum(acc_ref[0:16, :], axis=-1)
        corr_s = jnp.sum(acc_ref[16:32, :], axis=-1)
        cnt = jnp.sum(acc_ref[32:48, :], axis=-1)
        ece_in = jnp.abs((conf_s - corr_s) / n_samples)
        ece_t = jnp.sum(jnp.where(cnt > 0, ece_in, 0.0))
        ece_ref[0, 0, :] = jnp.full((128,), ece_t, jnp.float32)


def kernel(logits, labels):
    T, N, C = logits.shape
    R = 4096
    while N % R != 0:
        R //= 2
    NB = N // R

    logits_t = jnp.transpose(logits, (0, 2, 1))

    out = pl.pallas_call(
        functools.partial(_ece_tc_kernel, nb_total=NB, n_samples=N),
        grid=(T, NB),
        in_specs=[
            pl.BlockSpec((1, 1, R), lambda t, nb: (nb, 0, 0)),
            pl.BlockSpec((1, C, R), lambda t, nb: (t, 0, nb)),
        ],
        out_specs=pl.BlockSpec((1, 1, 128), lambda t, nb: (t, 0, 0)),
        out_shape=jax.ShapeDtypeStruct((T, 1, 128), jnp.float32),
        scratch_shapes=[pltpu.VMEM((48, R), jnp.float32)],
    )(labels.reshape(NB, 1, R), logits_t)
    return out[:, 0, 0]
